# baseline (speedup 1.0000x reference)
import functools

import jax
import jax.numpy as jnp
from jax import lax
from jax.experimental import pallas as pl
from jax.experimental.pallas import tpu as pltpu
from jax.experimental.pallas import tpu_sc as plsc

NC = 2
NS = 16
NW = NC * NS

HT = 25
BT = 32
N_BLOCKS = HT * BT
BLOCKS_PER_W = N_BLOCKS // NW


@jax.jit
def _emb_lookup(idx4, table):
    mesh = plsc.VectorSubcoreMesh(core_axis_name="c", subcore_axis_name="s")

    @functools.partial(
        pl.kernel,
        out_type=jax.ShapeDtypeStruct((200, 4, 32, 8, 128), jnp.float32),
        mesh=mesh,
        scratch_types=[
            pltpu.VMEM((8, 128), jnp.int32),
            pltpu.VMEM((1024, 32), jnp.float32),
            pltpu.VMEM((8, 4, 8, 128), jnp.float32),
            pltpu.SemaphoreType.DMA,
            pltpu.SemaphoreType.DMA,
        ],
        compiler_params=pltpu.CompilerParams(
            use_tc_tiling_on_sc=False, needs_layout_passes=False
        ),
    )
    def body(idx_hbm, table_hbm, out_hbm, idx_v, rows_v, ot_v, gsem, osem):
        wid = lax.axis_index("s") * NC + lax.axis_index("c")
        lane = lax.iota(jnp.int32, 16)

        def block_body(k, carry):
            blk = wid * BLOCKS_PER_W + k
            ih = blk // BT
            jb = blk % BT

            pltpu.sync_copy(idx_hbm.at[ih, jb], idx_v)
            copies = [
                pltpu.async_copy(
                    table_hbm.at[idx_v.at[hr]],
                    rows_v.at[pl.ds(hr * 128, 128)],
                    gsem,
                )
                for hr in range(8)
            ]
            for c in copies:
                c.wait()

            def tp_body(t, tc):
                hr = t // 32
                col = t % 32
                i = col // 8
                r = col % 8
                cvec = jnp.full((16,), col, jnp.int32)
                for s in range(8):
                    rvec = hr * 128 + s * 16 + lane
                    vals = plsc.load_gather(rows_v, [rvec, cvec])
                    ot_v[hr, i, r, pl.ds(s * 16, 16)] = vals
                return tc

            lax.fori_loop(0, 256, tp_body, 0)

            wcopies = [
                pltpu.async_copy(
                    ot_v.at[hr, i],
                    out_hbm.at[ih * 8 + hr, i, jb],
                    osem,
                )
                for hr in range(8)
                for i in range(4)
            ]
            for c in wcopies:
                c.wait()
            return carry

        lax.fori_loop(0, BLOCKS_PER_W, block_body, 0)

    return body(idx4, table)


def kernel(inputs, emb_table):
    batch, hist = inputs.shape
    emb_dim = emb_table.shape[1]
    idx = inputs.astype(jnp.int32)
    idx4 = idx.reshape(BT, 128, HT, 8).transpose(2, 0, 3, 1)
    out5 = _emb_lookup(idx4, emb_table)
    out = out5.transpose(2, 4, 0, 1, 3).reshape(batch, hist, emb_dim)
    return out


# speedup vs baseline: 1.0527x; 1.0527x over previous
import functools

import jax
import jax.numpy as jnp
from jax import lax
from jax.experimental import pallas as pl
from jax.experimental.pallas import tpu as pltpu
from jax.experimental.pallas import tpu_sc as plsc

NC = 2
NS = 16
NW = NC * NS

HT = 25
BT = 32
N_BLOCKS = HT * BT
BPW = N_BLOCKS // NW
PITCH = 32


@jax.jit
def _emb_lookup(idx4, table):
    mesh = plsc.VectorSubcoreMesh(core_axis_name="c", subcore_axis_name="s")

    @functools.partial(
        pl.kernel,
        out_type=jax.ShapeDtypeStruct((200, 4, 32, 8, 128), jnp.float32),
        mesh=mesh,
        scratch_types=[
            pltpu.VMEM((8, 128), jnp.int32),
            pltpu.VMEM((8, 128), jnp.int32),
            pltpu.VMEM((1024, PITCH), jnp.float32),
            pltpu.VMEM((1024, PITCH), jnp.float32),
            pltpu.VMEM((8, 4, 8, 128), jnp.float32),
            pltpu.SemaphoreType.DMA,
            pltpu.SemaphoreType.DMA,
            pltpu.SemaphoreType.DMA,
        ],
        compiler_params=pltpu.CompilerParams(
            use_tc_tiling_on_sc=False, needs_layout_passes=False
        ),
    )
    def body(idx_hbm, table_hbm, out_hbm, idx_v0, idx_v1, rows_v0, rows_v1,
             ot_v, gsem0, gsem1, osem):
        wid = lax.axis_index("s") * NC + lax.axis_index("c")
        blk0 = wid * BPW
        lane = lax.iota(jnp.int32, 16)

        def fire(blk, idx_v, rows_v, gsem):
            ih = blk // BT
            jb = blk % BT
            pltpu.sync_copy(idx_hbm.at[ih, jb], idx_v)
            for hr in range(8):
                pltpu.async_copy(
                    table_hbm.at[idx_v.at[hr]],
                    rows_v.at[pl.ds(hr * 128, 128)],
                    gsem,
                )

        def wait_gathers(idx_v, rows_v, gsem):
            for hr in range(8):
                pltpu.make_async_copy(
                    table_hbm.at[idx_v.at[hr]],
                    rows_v.at[pl.ds(hr * 128, 128)],
                    gsem,
                ).wait()

        def drain_writes():
            for _ in range(32):
                pltpu.make_async_copy(
                    ot_v.at[0, 0], out_hbm.at[0, 0, 0], osem
                ).wait()

        def transpose_and_write(blk, idx_v, rows_v, gsem, first):
            ih = blk // BT
            jb = blk % BT
            wait_gathers(idx_v, rows_v, gsem)
            if not first:
                drain_writes()

            def tp_body(t, tc):
                hr = t // 4
                i = t % 4
                base = hr * 128
                for r in range(8):
                    cvec = jnp.full((16,), 8 * i + r, jnp.int32)
                    for s in range(8):
                        rvec = base + s * 16 + lane
                        vals = plsc.load_gather(rows_v, [rvec, cvec])
                        ot_v[hr, i, r, pl.ds(s * 16, 16)] = vals
                return tc

            lax.fori_loop(0, 32, tp_body, 0)
            for hr in range(8):
                for i in range(4):
                    pltpu.async_copy(
                        ot_v.at[hr, i], out_hbm.at[ih * 8 + hr, i, jb], osem
                    )

        fire(blk0, idx_v0, rows_v0, gsem0)

        def pair(j, carry):
            b0 = blk0 + 2 * j
            fire(b0 + 1, idx_v1, rows_v1, gsem1)
            transpose_and_write(b0, idx_v0, rows_v0, gsem0, False)
            fire(b0 + 2, idx_v0, rows_v0, gsem0)
            transpose_and_write(b0 + 1, idx_v1, rows_v1, gsem1, False)
            return carry

        fire(blk0 + 1, idx_v1, rows_v1, gsem1)
        transpose_and_write(blk0, idx_v0, rows_v0, gsem0, True)
        fire(blk0 + 2, idx_v0, rows_v0, gsem0)
        transpose_and_write(blk0 + 1, idx_v1, rows_v1, gsem1, False)
        lax.fori_loop(1, 12, pair, 0)
        transpose_and_write(blk0 + 24, idx_v0, rows_v0, gsem0, False)
        drain_writes()

    return body(idx4, table)


def kernel(inputs, emb_table):
    batch, hist = inputs.shape
    emb_dim = emb_table.shape[1]
    idx = inputs.astype(jnp.int32)
    idx4 = idx.reshape(BT, 128, HT, 8).transpose(2, 0, 3, 1)
    out5 = _emb_lookup(idx4, emb_table)
    out = out5.transpose(2, 4, 0, 1, 3).reshape(batch, hist, emb_dim)
    return out


# speedup vs baseline: 1.3004x; 1.2353x over previous
import functools

import jax
import jax.numpy as jnp
from jax import lax
from jax.experimental import pallas as pl
from jax.experimental.pallas import tpu as pltpu
from jax.experimental.pallas import tpu_sc as plsc

NC = 2
NS = 16
NW = NC * NS

HT = 25
BT = 32
N_BLOCKS = HT * BT
BPW = N_BLOCKS // NW
PITCH = 32


@jax.jit
def _emb_lookup(idx4, table):
    mesh = plsc.VectorSubcoreMesh(core_axis_name="c", subcore_axis_name="s")

    @functools.partial(
        pl.kernel,
        out_type=jax.ShapeDtypeStruct((200, 4, 32, 8, 128), jnp.float32),
        mesh=mesh,
        scratch_types=[
            pltpu.VMEM((8, 128), jnp.int32),
            pltpu.VMEM((8, 128), jnp.int32),
            pltpu.VMEM((1024, PITCH), jnp.float32),
            pltpu.VMEM((1024, PITCH), jnp.float32),
            pltpu.VMEM((8, 4, 8, 128), jnp.float32),
            pltpu.SemaphoreType.DMA,
            pltpu.SemaphoreType.DMA,
            pltpu.SemaphoreType.DMA,
        ],
        compiler_params=pltpu.CompilerParams(
            use_tc_tiling_on_sc=False, needs_layout_passes=False
        ),
    )
    def body(idx_hbm, table_hbm, out_hbm, idx_v0, idx_v1, rows_v0, rows_v1,
             ot_v, gsem0, gsem1, osem):
        wid = lax.axis_index("s") * NC + lax.axis_index("c")
        blk0 = wid * BPW
        lane = lax.iota(jnp.int32, 16)

        def fire(blk, idx_v, rows_v, gsem):
            ih = blk // BT
            jb = blk % BT
            pltpu.sync_copy(idx_hbm.at[ih, jb], idx_v)
            for hr in range(8):
                pltpu.async_copy(
                    table_hbm.at[idx_v.at[hr]],
                    rows_v.at[pl.ds(hr * 128, 128)],
                    gsem,
                )

        def wait_gathers(idx_v, rows_v, gsem):
            for hr in range(8):
                pltpu.make_async_copy(
                    table_hbm.at[idx_v.at[hr]],
                    rows_v.at[pl.ds(hr * 128, 128)],
                    gsem,
                ).wait()

        def drain_writes():
            for _ in range(32):
                pltpu.make_async_copy(
                    ot_v.at[0, 0], out_hbm.at[0, 0, 0], osem
                ).wait()

        def transpose_and_write(blk, idx_v, rows_v, gsem, first):
            ih = blk // BT
            jb = blk % BT
            wait_gathers(idx_v, rows_v, gsem)
            if not first:
                drain_writes()

            @plsc.parallel_loop(0, 32, 1, unroll=2)
            def tp_body(t):
                hr = t // 4
                i = t % 4
                base = hr * 128
                for r in range(8):
                    cvec = jnp.full((16,), 8 * i + r, jnp.int32)
                    for s in range(8):
                        rvec = base + s * 16 + lane
                        vals = plsc.load_gather(rows_v, [rvec, cvec])
                        ot_v[hr, i, r, pl.ds(s * 16, 16)] = vals
            for hr in range(8):
                for i in range(4):
                    pltpu.async_copy(
                        ot_v.at[hr, i], out_hbm.at[ih * 8 + hr, i, jb], osem
                    )

        fire(blk0, idx_v0, rows_v0, gsem0)

        def pair(j, carry):
            b0 = blk0 + 2 * j
            fire(b0 + 1, idx_v1, rows_v1, gsem1)
            transpose_and_write(b0, idx_v0, rows_v0, gsem0, False)
            fire(b0 + 2, idx_v0, rows_v0, gsem0)
            transpose_and_write(b0 + 1, idx_v1, rows_v1, gsem1, False)
            return carry

        fire(blk0 + 1, idx_v1, rows_v1, gsem1)
        transpose_and_write(blk0, idx_v0, rows_v0, gsem0, True)
        fire(blk0 + 2, idx_v0, rows_v0, gsem0)
        transpose_and_write(blk0 + 1, idx_v1, rows_v1, gsem1, False)
        lax.fori_loop(1, 12, pair, 0)
        transpose_and_write(blk0 + 24, idx_v0, rows_v0, gsem0, False)
        drain_writes()

    return body(idx4, table)


def kernel(inputs, emb_table):
    batch, hist = inputs.shape
    emb_dim = emb_table.shape[1]
    idx = inputs.astype(jnp.int32)
    idx4 = idx.reshape(BT, 128, HT, 8).transpose(2, 0, 3, 1)
    out5 = _emb_lookup(idx4, emb_table)
    out = out5.transpose(2, 4, 0, 1, 3).reshape(batch, hist, emb_dim)
    return out


# speedup vs baseline: 1.3829x; 1.0635x over previous
import functools

import jax
import jax.numpy as jnp
from jax import lax
from jax.experimental import pallas as pl
from jax.experimental.pallas import tpu as pltpu
from jax.experimental.pallas import tpu_sc as plsc

NC = 2
NS = 16
NW = NC * NS

HT = 25
BT = 32
N_BLOCKS = HT * BT
BPW = N_BLOCKS // NW
PITCH = 32


@jax.jit
def _emb_lookup(idx4, table):
    mesh = plsc.VectorSubcoreMesh(core_axis_name="c", subcore_axis_name="s")

    @functools.partial(
        pl.kernel,
        out_type=jax.ShapeDtypeStruct((200, 4, 32, 8, 128), jnp.float32),
        mesh=mesh,
        scratch_types=[
            pltpu.VMEM((8, 128), jnp.int32),
            pltpu.VMEM((8, 128), jnp.int32),
            pltpu.VMEM((1024, PITCH), jnp.float32),
            pltpu.VMEM((1024, PITCH), jnp.float32),
            pltpu.VMEM((8, 4, 8, 128), jnp.float32),
            pltpu.SemaphoreType.DMA,
            pltpu.SemaphoreType.DMA,
            pltpu.SemaphoreType.DMA,
        ],
        compiler_params=pltpu.CompilerParams(
            use_tc_tiling_on_sc=False, needs_layout_passes=False
        ),
    )
    def body(idx_hbm, table_hbm, out_hbm, idx_v0, idx_v1, rows_v0, rows_v1,
             ot_v, gsem0, gsem1, osem):
        wid = lax.axis_index("s") * NC + lax.axis_index("c")
        blk0 = wid * BPW
        lane = lax.iota(jnp.int32, 16)

        def fire(blk, idx_v, rows_v, gsem):
            ih = blk // BT
            jb = blk % BT
            pltpu.sync_copy(idx_hbm.at[ih, jb], idx_v)
            for hr in range(8):
                pltpu.async_copy(
                    table_hbm.at[idx_v.at[hr]],
                    rows_v.at[pl.ds(hr * 128, 128)],
                    gsem,
                )

        def wait_gathers(idx_v, rows_v, gsem):
            for hr in range(8):
                pltpu.make_async_copy(
                    table_hbm.at[idx_v.at[hr]],
                    rows_v.at[pl.ds(hr * 128, 128)],
                    gsem,
                ).wait()

        def drain_writes():
            for _ in range(32):
                pltpu.make_async_copy(
                    ot_v.at[0, 0], out_hbm.at[0, 0, 0], osem
                ).wait()

        def transpose_and_write(blk, idx_v, rows_v, gsem, first):
            ih = blk // BT
            jb = blk % BT
            wait_gathers(idx_v, rows_v, gsem)
            if not first:
                drain_writes()

            @plsc.parallel_loop(0, 32, 1, unroll=4)
            def tp_body(t):
                hr = t // 4
                i = t % 4
                base = hr * 128
                for r in range(8):
                    cvec = jnp.full((16,), 8 * i + r, jnp.int32)
                    for s in range(8):
                        rvec = base + s * 16 + lane
                        vals = plsc.load_gather(rows_v, [rvec, cvec])
                        ot_v[hr, i, r, pl.ds(s * 16, 16)] = vals
            for hr in range(8):
                for i in range(4):
                    pltpu.async_copy(
                        ot_v.at[hr, i], out_hbm.at[ih * 8 + hr, i, jb], osem
                    )

        fire(blk0, idx_v0, rows_v0, gsem0)

        def pair(j, carry):
            b0 = blk0 + 2 * j
            fire(b0 + 1, idx_v1, rows_v1, gsem1)
            transpose_and_write(b0, idx_v0, rows_v0, gsem0, False)
            fire(b0 + 2, idx_v0, rows_v0, gsem0)
            transpose_and_write(b0 + 1, idx_v1, rows_v1, gsem1, False)
            return carry

        fire(blk0 + 1, idx_v1, rows_v1, gsem1)
        transpose_and_write(blk0, idx_v0, rows_v0, gsem0, True)
        fire(blk0 + 2, idx_v0, rows_v0, gsem0)
        transpose_and_write(blk0 + 1, idx_v1, rows_v1, gsem1, False)
        lax.fori_loop(1, 12, pair, 0)
        transpose_and_write(blk0 + 24, idx_v0, rows_v0, gsem0, False)
        drain_writes()

    return body(idx4, table)


def kernel(inputs, emb_table):
    batch, hist = inputs.shape
    emb_dim = emb_table.shape[1]
    idx = inputs.astype(jnp.int32)
    idx4 = idx.reshape(BT, 128, HT, 8).transpose(2, 0, 3, 1)
    out5 = _emb_lookup(idx4, emb_table)
    out = out5.transpose(2, 4, 0, 1, 3).reshape(batch, hist, emb_dim)
    return out
